# baseline (speedup 1.0000x reference)
import functools

import jax
import jax.numpy as jnp
from jax import lax
from jax.experimental import pallas as pl
from jax.experimental.pallas import tpu as pltpu
from jax.experimental.pallas import tpu_sc as plsc

B = 1024
K = 1000
D = 128
OUT = 64
TOP_K = 10
KP = 1024
BB = 256
NW = 32
RPW = B // NW
NCH = KP // 16
ROWG = 4

_NEG = -3.0e38


def _lp_body(x_ref, mean_ref, stddev_ref, lp_ref, cid_ref):
    x = x_ref[...]
    mean = mean_ref[...]
    std = stddev_ref[...]

    iv = 1.0 / (std * std)
    w2 = mean * iv
    bias = (jnp.sum(-0.5 * mean * w2 - jnp.log(std), axis=1)
            - 0.5 * D * jnp.log(2.0 * jnp.pi))

    t1 = jax.lax.dot_general(x * x, iv, (((1,), (1,)), ((), ())),
                             preferred_element_type=jnp.float32,
                             precision=jax.lax.Precision.HIGHEST)
    t2 = jax.lax.dot_general(x, w2, (((1,), (1,)), ((), ())),
                             preferred_element_type=jnp.float32,
                             precision=jax.lax.Precision.HIGHEST)
    lp = -0.5 * t1 + t2 + bias[None, :]
    lp = jnp.concatenate(
        [lp, jnp.full((BB, KP - K), _NEG, jnp.float32)], axis=1)
    lp_ref[...] = lp

    m = jnp.max(lp.reshape(BB, NCH, 16), axis=2)
    iota = jax.lax.broadcasted_iota(jnp.int32, (BB, NCH), 1)
    col = jax.lax.broadcasted_iota(jnp.int32, (BB, 128), 1)
    cid = jnp.zeros((BB, 128), jnp.int32)
    for i in range(TOP_K):
        cur = jnp.max(m, axis=1, keepdims=True)
        ismax = m == cur
        first = jnp.min(jnp.where(ismax, iota, NCH), axis=1, keepdims=True)
        cid = jnp.where(col == i, first, cid)
        m = jnp.where(iota == first, _NEG, m)
    cid_ref[...] = cid


def _distances(x, mean, stddev):
    return pl.pallas_call(
        _lp_body,
        grid=(B // BB,),
        in_specs=[
            pl.BlockSpec((BB, D), lambda i: (i, 0)),
            pl.BlockSpec((K, D), lambda i: (0, 0)),
            pl.BlockSpec((K, D), lambda i: (0, 0)),
        ],
        out_specs=[
            pl.BlockSpec((BB, KP), lambda i: (i, 0)),
            pl.BlockSpec((BB, 128), lambda i: (i, 0)),
        ],
        out_shape=[
            jax.ShapeDtypeStruct((B, KP), jnp.float32),
            jax.ShapeDtypeStruct((B, 128), jnp.int32),
        ],
        compiler_params=pltpu.CompilerParams(
            dimension_semantics=("arbitrary",)),
    )(x, mean, stddev)


def _lane_bcast(v, j):
    idx = jnp.full((16, 1), j, jnp.int32)
    dn = lax.GatherDimensionNumbers(
        offset_dims=(), collapsed_slice_dims=(0,), start_index_map=(0,))
    return lax.gather(v, idx, dn, (1,),
                      mode=lax.GatherScatterMode.PROMISE_IN_BOUNDS)


def _make_sc_combine():
    mesh = plsc.VectorSubcoreMesh(core_axis_name="c", subcore_axis_name="s",
                                  num_cores=2, num_subcores=16)

    @functools.partial(
        pl.kernel,
        out_type=jax.ShapeDtypeStruct((B, OUT), jnp.float32),
        mesh=mesh,
        scratch_types=[
            pltpu.VMEM((RPW, KP), jnp.float32),
            pltpu.VMEM((RPW, 128), jnp.int32),
            pltpu.VMEM((RPW * 16,), jnp.int32),
            pltpu.VMEM((RPW, 16), jnp.float32),
            pltpu.VMEM((RPW * 16, 128), jnp.float32),
            pltpu.VMEM((RPW, OUT), jnp.float32),
            pltpu.SemaphoreType.DMA,
        ],
        compiler_params=pltpu.CompilerParams(needs_layout_passes=False),
    )
    def sc_top10(lp_hbm, cid_hbm, outputs_hbm, out_hbm,
                 rows_v, cid_v, idx_v, w_v, gath_v, acc_v, sem):
        wid = lax.axis_index("s") * 2 + lax.axis_index("c")
        base = pl.multiple_of(wid * RPW, RPW)
        pltpu.sync_copy(lp_hbm.at[pl.ds(base, RPW)], rows_v)
        pltpu.sync_copy(cid_hbm.at[pl.ds(base, RPW)], cid_v)

        lane = lax.broadcasted_iota(jnp.int32, (16,), 0)

        def merge_rows(g, _):
            gbase = pl.multiple_of(g * ROWG, ROWG)
            for q in range(ROWG):
                r = gbase + q
                rfull = jnp.full((16,), r, jnp.int32)
                cid = cid_v[r, pl.ds(0, 16)]
                c0 = _lane_bcast(cid, 0)
                gi = c0 * 16 + lane
                v = plsc.load_gather(rows_v, [rfull, gi])
                tv, ti = plsc.sort_key_val(v, gi)
                for j in range(1, TOP_K):
                    cj = _lane_bcast(cid, j)
                    gi = cj * 16 + lane
                    v = plsc.load_gather(rows_v, [rfull, gi])
                    sv, si = plsc.sort_key_val(v, gi, descending=True)
                    m = tv >= sv
                    nv = jnp.where(m, tv, sv)
                    ni = jnp.where(m, ti, si)
                    tv, ti = plsc.sort_key_val(nv, ni)
                m0 = jnp.max(tv)
                e = jnp.where(lane >= 16 - TOP_K, jnp.exp(tv - m0), 0.0)
                w_v[r] = e / jnp.sum(e)
                idx_v[pl.ds(pl.multiple_of(r * 16, 16), 16)] = (
                    jnp.minimum(ti, K - 1))
            return 0

        lax.fori_loop(0, RPW // ROWG, merge_rows, 0)

        pltpu.async_copy(outputs_hbm.at[idx_v], gath_v, sem).wait()

        def crow(r, _):
            wvec = w_v[r]
            for c in range(OUT // 16):
                acc = jnp.zeros((16,), jnp.float32)
                for j in range(16 - TOP_K, 16):
                    wj = _lane_bcast(wvec, j)
                    acc = acc + wj * gath_v[r * 16 + j, pl.ds(c * 16, 16)]
                acc_v[r, pl.ds(c * 16, 16)] = acc
            return 0

        lax.fori_loop(0, RPW, crow, 0)
        pltpu.sync_copy(acc_v, out_hbm.at[pl.ds(base, RPW)])

    return sc_top10


_sc_combine = _make_sc_combine()


@jax.jit
def kernel(x, mean, stddev, outputs):
    lp, cid = _distances(x, mean, stddev)
    outputs_p = jnp.concatenate(
        [outputs, jnp.zeros((K, 128 - OUT), jnp.float32)], axis=1)
    return _sc_combine(lp, cid, outputs_p)


# speedup vs baseline: 1.0005x; 1.0005x over previous
import functools

import jax
import jax.numpy as jnp
from jax import lax
from jax.experimental import pallas as pl
from jax.experimental.pallas import tpu as pltpu
from jax.experimental.pallas import tpu_sc as plsc

B = 1024
K = 1000
D = 128
OUT = 64
TOP_K = 10
KP = 1024
BB = 256
NW = 32
RPW = B // NW
NCH = KP // 16
ROWG = 4

_NEG = -3.0e38


def _lp_body(x_ref, mean_ref, stddev_ref, lp_ref, cid_ref):
    x = x_ref[...]
    mean = mean_ref[...]
    std = stddev_ref[...]

    iv = 1.0 / (std * std)
    w2 = mean * iv
    bias = (jnp.sum(-0.5 * mean * w2 - jnp.log(std), axis=1)
            - 0.5 * D * jnp.log(2.0 * jnp.pi))

    t1 = jax.lax.dot_general(x * x, iv, (((1,), (1,)), ((), ())),
                             preferred_element_type=jnp.float32,
                             precision=jax.lax.Precision.HIGHEST)
    t2 = jax.lax.dot_general(x, w2, (((1,), (1,)), ((), ())),
                             preferred_element_type=jnp.float32,
                             precision=jax.lax.Precision.HIGHEST)
    lp = -0.5 * t1 + t2 + bias[None, :]
    lp = jnp.concatenate(
        [lp, jnp.full((BB, KP - K), _NEG, jnp.float32)], axis=1)
    lp_ref[...] = lp

    m = jnp.max(lp.reshape(BB, NCH, 16), axis=2)
    iota = jax.lax.broadcasted_iota(jnp.int32, (BB, NCH), 1)
    col = jax.lax.broadcasted_iota(jnp.int32, (BB, 128), 1)
    cid = jnp.zeros((BB, 128), jnp.int32)
    for i in range(TOP_K):
        cur = jnp.max(m, axis=1, keepdims=True)
        ismax = m == cur
        first = jnp.min(jnp.where(ismax, iota, NCH), axis=1, keepdims=True)
        cid = jnp.where(col == i, first, cid)
        m = jnp.where(iota == first, _NEG, m)
    cid_ref[...] = cid


def _distances(x, mean, stddev):
    return pl.pallas_call(
        _lp_body,
        grid=(B // BB,),
        in_specs=[
            pl.BlockSpec((BB, D), lambda i: (i, 0)),
            pl.BlockSpec((K, D), lambda i: (0, 0)),
            pl.BlockSpec((K, D), lambda i: (0, 0)),
        ],
        out_specs=[
            pl.BlockSpec((BB, KP), lambda i: (i, 0)),
            pl.BlockSpec((BB, 128), lambda i: (i, 0)),
        ],
        out_shape=[
            jax.ShapeDtypeStruct((B, KP), jnp.float32),
            jax.ShapeDtypeStruct((B, 128), jnp.int32),
        ],
        compiler_params=pltpu.CompilerParams(
            dimension_semantics=("arbitrary",)),
    )(x, mean, stddev)


def _lane_bcast(v, j):
    idx = jnp.full((16, 1), j, jnp.int32)
    dn = lax.GatherDimensionNumbers(
        offset_dims=(), collapsed_slice_dims=(0,), start_index_map=(0,))
    return lax.gather(v, idx, dn, (1,),
                      mode=lax.GatherScatterMode.PROMISE_IN_BOUNDS)


def _make_sc_combine():
    mesh = plsc.VectorSubcoreMesh(core_axis_name="c", subcore_axis_name="s",
                                  num_cores=2, num_subcores=16)

    @functools.partial(
        pl.kernel,
        out_type=jax.ShapeDtypeStruct((B, OUT), jnp.float32),
        mesh=mesh,
        scratch_types=[
            pltpu.VMEM((RPW, KP), jnp.float32),
            pltpu.VMEM((RPW, 128), jnp.int32),
            pltpu.VMEM((RPW * 16,), jnp.int32),
            pltpu.VMEM((RPW, 16), jnp.float32),
            pltpu.VMEM((RPW * 16, 128), jnp.float32),
            pltpu.VMEM((RPW, OUT), jnp.float32),
            pltpu.SemaphoreType.DMA,
        ],
        compiler_params=pltpu.CompilerParams(needs_layout_passes=False),
    )
    def sc_top10(lp_hbm, cid_hbm, outputs_hbm, out_hbm,
                 rows_v, cid_v, idx_v, w_v, gath_v, acc_v, sem):
        wid = lax.axis_index("s") * 2 + lax.axis_index("c")
        base = pl.multiple_of(wid * RPW, RPW)
        with jax.named_scope("sc_dma_in"):
            pltpu.sync_copy(lp_hbm.at[pl.ds(base, RPW)], rows_v)
            pltpu.sync_copy(cid_hbm.at[pl.ds(base, RPW)], cid_v)

        lane = lax.broadcasted_iota(jnp.int32, (16,), 0)

        def merge_rows(g, _):
            gbase = pl.multiple_of(g * ROWG, ROWG)
            for q in range(ROWG):
                r = gbase + q
                rfull = jnp.full((16,), r, jnp.int32)
                cid = cid_v[r, pl.ds(0, 16)]
                c0 = _lane_bcast(cid, 0)
                gi = c0 * 16 + lane
                v = plsc.load_gather(rows_v, [rfull, gi])
                tv, ti = plsc.sort_key_val(v, gi)
                for j in range(1, TOP_K):
                    cj = _lane_bcast(cid, j)
                    gi = cj * 16 + lane
                    v = plsc.load_gather(rows_v, [rfull, gi])
                    sv, si = plsc.sort_key_val(v, gi, descending=True)
                    m = tv >= sv
                    nv = jnp.where(m, tv, sv)
                    ni = jnp.where(m, ti, si)
                    tv, ti = plsc.sort_key_val(nv, ni)
                m0 = jnp.max(tv)
                e = jnp.where(lane >= 16 - TOP_K, jnp.exp(tv - m0), 0.0)
                w_v[r] = e / jnp.sum(e)
                idx_v[pl.ds(pl.multiple_of(r * 16, 16), 16)] = (
                    jnp.minimum(ti, K - 1))
            return 0

        with jax.named_scope("sc_merge"):
            lax.fori_loop(0, RPW // ROWG, merge_rows, 0)

        with jax.named_scope("sc_gather"):
            pltpu.async_copy(outputs_hbm.at[idx_v], gath_v, sem).wait()

        def crow(r, _):
            wvec = w_v[r]
            for c in range(OUT // 16):
                acc = jnp.zeros((16,), jnp.float32)
                for j in range(16 - TOP_K, 16):
                    wj = _lane_bcast(wvec, j)
                    acc = acc + wj * gath_v[r * 16 + j, pl.ds(c * 16, 16)]
                acc_v[r, pl.ds(c * 16, 16)] = acc
            return 0

        with jax.named_scope("sc_combine"):
            lax.fori_loop(0, RPW, crow, 0)
        with jax.named_scope("sc_dma_out"):
            pltpu.sync_copy(acc_v, out_hbm.at[pl.ds(base, RPW)])

    return sc_top10


_sc_combine = _make_sc_combine()


@jax.jit
def kernel(x, mean, stddev, outputs):
    lp, cid = _distances(x, mean, stddev)
    outputs_p = jnp.concatenate(
        [outputs, jnp.zeros((K, 128 - OUT), jnp.float32)], axis=1)
    return _sc_combine(lp, cid, outputs_p)


# speedup vs baseline: 1.4539x; 1.4531x over previous
import functools

import jax
import jax.numpy as jnp
from jax import lax
from jax.experimental import pallas as pl
from jax.experimental.pallas import tpu as pltpu
from jax.experimental.pallas import tpu_sc as plsc

B = 1024
K = 1000
D = 128
OUT = 64
TOP_K = 10
KP = 1024
BB = 256
NW = 32
RPW = B // NW
NG = 128
GSZ = KP // NG
ROWG = 4

_NEG = -3.0e38


def _lp_body(x_ref, mean_ref, stddev_ref, lp_ref, gid_ref):
    x = x_ref[...]
    mean = mean_ref[...]
    std = stddev_ref[...]

    iv = 1.0 / (std * std)
    w2 = mean * iv
    bias = (jnp.sum(-0.5 * mean * w2 - jnp.log(std), axis=1)
            - 0.5 * D * jnp.log(2.0 * jnp.pi))

    t1 = jax.lax.dot_general(x * x, iv, (((1,), (1,)), ((), ())),
                             preferred_element_type=jnp.float32,
                             precision=jax.lax.Precision.HIGHEST)
    t2 = jax.lax.dot_general(x, w2, (((1,), (1,)), ((), ())),
                             preferred_element_type=jnp.float32,
                             precision=jax.lax.Precision.HIGHEST)
    lp = -0.5 * t1 + t2 + bias[None, :]
    lp = jnp.concatenate(
        [lp, jnp.full((BB, KP - K), _NEG, jnp.float32)], axis=1)
    lp_ref[...] = lp

    m = jnp.max(lp.reshape(BB, GSZ, NG), axis=1)
    iota = jax.lax.broadcasted_iota(jnp.int32, (BB, NG), 1)
    gid = jnp.zeros((BB, NG), jnp.int32)
    for i in range(TOP_K):
        cur = jnp.max(m, axis=1, keepdims=True)
        ismax = m == cur
        first = jnp.min(jnp.where(ismax, iota, NG), axis=1, keepdims=True)
        gid = jnp.where(iota == i, first, gid)
        m = jnp.where(iota == first, _NEG, m)
    gid_ref[...] = gid


def _distances(x, mean, stddev):
    return pl.pallas_call(
        _lp_body,
        grid=(B // BB,),
        in_specs=[
            pl.BlockSpec((BB, D), lambda i: (i, 0)),
            pl.BlockSpec((K, D), lambda i: (0, 0)),
            pl.BlockSpec((K, D), lambda i: (0, 0)),
        ],
        out_specs=[
            pl.BlockSpec((BB, KP), lambda i: (i, 0)),
            pl.BlockSpec((BB, NG), lambda i: (i, 0)),
        ],
        out_shape=[
            jax.ShapeDtypeStruct((B, KP), jnp.float32),
            jax.ShapeDtypeStruct((B, NG), jnp.int32),
        ],
        compiler_params=pltpu.CompilerParams(
            dimension_semantics=("arbitrary",)),
    )(x, mean, stddev)


def _lane_bcast(v, j):
    idx = jnp.full((16, 1), j, jnp.int32)
    dn = lax.GatherDimensionNumbers(
        offset_dims=(), collapsed_slice_dims=(0,), start_index_map=(0,))
    return lax.gather(v, idx, dn, (1,),
                      mode=lax.GatherScatterMode.PROMISE_IN_BOUNDS)


def _make_sc_combine():
    mesh = plsc.VectorSubcoreMesh(core_axis_name="c", subcore_axis_name="s",
                                  num_cores=2, num_subcores=16)

    @functools.partial(
        pl.kernel,
        out_type=jax.ShapeDtypeStruct((B, OUT), jnp.float32),
        mesh=mesh,
        scratch_types=[
            pltpu.VMEM((RPW, KP), jnp.float32),
            pltpu.VMEM((RPW, NG), jnp.int32),
            pltpu.VMEM((RPW * 16,), jnp.int32),
            pltpu.VMEM((RPW, 16), jnp.float32),
            pltpu.VMEM((K // 2, 2 * OUT), jnp.float32),
            pltpu.VMEM((RPW, OUT), jnp.float32),
            pltpu.SemaphoreType.DMA,
        ],
        compiler_params=pltpu.CompilerParams(needs_layout_passes=False),
    )
    def sc_top10(lp_hbm, gid_hbm, outputs_hbm, out_hbm,
                 rows_v, gid_v, idx_v, w_v, table_v, acc_v, sem):
        wid = lax.axis_index("s") * 2 + lax.axis_index("c")
        base = pl.multiple_of(wid * RPW, RPW)
        table_cp = pltpu.async_copy(outputs_hbm, table_v, sem)
        with jax.named_scope("sc_dma_in"):
            pltpu.sync_copy(lp_hbm.at[pl.ds(base, RPW)], rows_v)
            pltpu.sync_copy(gid_hbm.at[pl.ds(base, RPW)], gid_v)

        lane = lax.broadcasted_iota(jnp.int32, (16,), 0)
        lane8 = jnp.bitwise_and(lane, GSZ - 1)

        def merge_rows(g, _):
            gbase = pl.multiple_of(g * ROWG, ROWG)
            for q in range(ROWG):
                r = gbase + q
                rfull = jnp.full((16,), r, jnp.int32)
                gids = gid_v[r, pl.ds(0, 16)]
                g0 = _lane_bcast(gids, 0)
                gi = g0 + NG * lane8
                v = plsc.load_gather(rows_v, [rfull, gi])
                v = jnp.where(lane < GSZ, v, _NEG)
                tv, ti = plsc.sort_key_val(v, gi)
                for j in range(1, TOP_K):
                    gj = _lane_bcast(gids, j)
                    gi = gj + NG * lane8
                    v = plsc.load_gather(rows_v, [rfull, gi])
                    v = jnp.where(lane < GSZ, v, _NEG)
                    sv, si = plsc.sort_key_val(v, gi, descending=True)
                    m = tv >= sv
                    nv = jnp.where(m, tv, sv)
                    ni = jnp.where(m, ti, si)
                    tv, ti = plsc.sort_key_val(nv, ni)
                m0 = jnp.max(tv)
                e = jnp.where(lane >= 16 - TOP_K, jnp.exp(tv - m0), 0.0)
                w_v[r] = e / jnp.sum(e)
                idx_v[pl.ds(pl.multiple_of(r * 16, 16), 16)] = (
                    jnp.minimum(ti, K - 1))
            return 0

        with jax.named_scope("sc_merge"):
            lax.fori_loop(0, RPW // ROWG, merge_rows, 0)

        with jax.named_scope("sc_table_wait"):
            table_cp.wait()

        def crow(r, _):
            tiv = idx_v[pl.ds(pl.multiple_of(r * 16, 16), 16)]
            wvec = w_v[r]
            accs = [jnp.zeros((16,), jnp.float32) for _ in range(OUT // 16)]
            for j in range(16 - TOP_K, 16):
                kj = _lane_bcast(tiv, j)
                wj = _lane_bcast(wvec, j)
                krow = lax.shift_right_logical(kj, 1)
                kcol = jnp.bitwise_and(kj, 1) * OUT
                for c in range(OUT // 16):
                    col = kcol + c * 16 + lane
                    v = plsc.load_gather(table_v, [krow, col])
                    accs[c] = accs[c] + wj * v
            for c in range(OUT // 16):
                acc_v[r, pl.ds(c * 16, 16)] = accs[c]
            return 0

        with jax.named_scope("sc_combine"):
            lax.fori_loop(0, RPW, crow, 0)
        with jax.named_scope("sc_dma_out"):
            pltpu.sync_copy(acc_v, out_hbm.at[pl.ds(base, RPW)])

    return sc_top10


_sc_combine = _make_sc_combine()


@jax.jit
def kernel(x, mean, stddev, outputs):
    lp, gid = _distances(x, mean, stddev)
    outputs2 = outputs.reshape(K // 2, 2 * OUT)
    return _sc_combine(lp, gid, outputs2)


# speedup vs baseline: 1.6140x; 1.1101x over previous
import functools

import jax
import jax.numpy as jnp
from jax import lax
from jax.experimental import pallas as pl
from jax.experimental.pallas import tpu as pltpu
from jax.experimental.pallas import tpu_sc as plsc

B = 1024
K = 1000
D = 128
OUT = 64
TOP_K = 10
KP = 1024
BB = 256
NW = 32
RPW = B // NW
NG = 128
GSZ = KP // NG
ROWG = 4

_NEG = -3.0e38


def _lp_body(x_ref, mean_ref, stddev_ref, lp_ref, gid_ref,
             w1_s, w2_s, bias_s):
    @pl.when(pl.program_id(0) == 0)
    def _prep():
        mean = mean_ref[...]
        std = stddev_ref[...]
        iv = 1.0 / (std * std)
        w2 = mean * iv
        w1_s[...] = -0.5 * iv
        w2_s[...] = w2
        bias_s[...] = (jnp.sum(-0.5 * mean * w2 - jnp.log(std), axis=1)
                       - 0.5 * D * jnp.log(2.0 * jnp.pi))[None, :]

    x = x_ref[...]
    t1 = jax.lax.dot_general(x * x, w1_s[...], (((1,), (1,)), ((), ())),
                             preferred_element_type=jnp.float32,
                             precision=jax.lax.Precision.HIGHEST)
    t2 = jax.lax.dot_general(x, w2_s[...], (((1,), (1,)), ((), ())),
                             preferred_element_type=jnp.float32,
                             precision=jax.lax.Precision.HIGHEST)
    lp = t1 + t2 + bias_s[...]
    lp = jnp.concatenate(
        [lp, jnp.full((BB, KP - K), _NEG, jnp.float32)], axis=1)
    lp_ref[...] = lp

    m = jnp.max(lp.reshape(BB, GSZ, NG), axis=1)
    iota = jax.lax.broadcasted_iota(jnp.int32, (BB, NG), 1)
    gid = jnp.zeros((BB, NG), jnp.int32)
    for i in range(TOP_K):
        cur = jnp.max(m, axis=1, keepdims=True)
        ismax = m == cur
        first = jnp.min(jnp.where(ismax, iota, NG), axis=1, keepdims=True)
        gid = jnp.where(iota == i, first, gid)
        m = jnp.where(iota == first, _NEG, m)
    gid_ref[...] = gid


def _distances(x, mean, stddev):
    return pl.pallas_call(
        _lp_body,
        grid=(B // BB,),
        in_specs=[
            pl.BlockSpec((BB, D), lambda i: (i, 0)),
            pl.BlockSpec((K, D), lambda i: (0, 0)),
            pl.BlockSpec((K, D), lambda i: (0, 0)),
        ],
        out_specs=[
            pl.BlockSpec((BB, KP), lambda i: (i, 0)),
            pl.BlockSpec((BB, NG), lambda i: (i, 0)),
        ],
        out_shape=[
            jax.ShapeDtypeStruct((B, KP), jnp.float32),
            jax.ShapeDtypeStruct((B, NG), jnp.int32),
        ],
        scratch_shapes=[
            pltpu.VMEM((K, D), jnp.float32),
            pltpu.VMEM((K, D), jnp.float32),
            pltpu.VMEM((1, K), jnp.float32),
        ],
        compiler_params=pltpu.CompilerParams(
            dimension_semantics=("arbitrary",)),
    )(x, mean, stddev)


def _lane_bcast(v, j):
    idx = jnp.full((16, 1), j, jnp.int32)
    dn = lax.GatherDimensionNumbers(
        offset_dims=(), collapsed_slice_dims=(0,), start_index_map=(0,))
    return lax.gather(v, idx, dn, (1,),
                      mode=lax.GatherScatterMode.PROMISE_IN_BOUNDS)


def _make_sc_combine():
    mesh = plsc.VectorSubcoreMesh(core_axis_name="c", subcore_axis_name="s",
                                  num_cores=2, num_subcores=16)

    @functools.partial(
        pl.kernel,
        out_type=jax.ShapeDtypeStruct((B, OUT), jnp.float32),
        mesh=mesh,
        scratch_types=[
            pltpu.VMEM((RPW, KP), jnp.float32),
            pltpu.VMEM((RPW, NG), jnp.int32),
            pltpu.VMEM((RPW * 16,), jnp.int32),
            pltpu.VMEM((RPW, 16), jnp.float32),
            pltpu.VMEM((K // 2, 2 * OUT), jnp.float32),
            pltpu.VMEM((RPW, OUT), jnp.float32),
            pltpu.SemaphoreType.DMA,
        ],
        compiler_params=pltpu.CompilerParams(needs_layout_passes=False),
    )
    def sc_top10(lp_hbm, gid_hbm, outputs_hbm, out_hbm,
                 rows_v, gid_v, idx_v, w_v, table_v, acc_v, sem):
        wid = lax.axis_index("s") * 2 + lax.axis_index("c")
        base = pl.multiple_of(wid * RPW, RPW)
        with jax.named_scope("sc_dma_in"):
            pltpu.sync_copy(lp_hbm.at[pl.ds(base, RPW)], rows_v)
            pltpu.sync_copy(gid_hbm.at[pl.ds(base, RPW)], gid_v)
        table_cp = pltpu.async_copy(outputs_hbm, table_v, sem)

        lane = lax.broadcasted_iota(jnp.int32, (16,), 0)
        lane8 = jnp.bitwise_and(lane, GSZ - 1)

        def merge_rows(g, _):
            gbase = pl.multiple_of(g * ROWG, ROWG)
            for q in range(ROWG):
                r = gbase + q
                rfull = jnp.full((16,), r, jnp.int32)
                gids = gid_v[r, pl.ds(0, 16)]
                hi = lane >= GSZ

                def pair(j2):
                    ga = _lane_bcast(gids, 2 * j2)
                    gb = _lane_bcast(gids, 2 * j2 + 1)
                    gi = jnp.where(hi, gb, ga) + NG * lane8
                    return plsc.load_gather(rows_v, [rfull, gi]), gi

                v, gi = pair(0)
                tv, ti = plsc.sort_key_val(v, gi)
                for j2 in range(1, TOP_K // 2):
                    v, gi = pair(j2)
                    sv, si = plsc.sort_key_val(v, gi, descending=True)
                    m = tv >= sv
                    nv = jnp.where(m, tv, sv)
                    ni = jnp.where(m, ti, si)
                    tv, ti = plsc.sort_key_val(nv, ni)
                m0 = jnp.max(tv)
                e = jnp.where(lane >= 16 - TOP_K, jnp.exp(tv - m0), 0.0)
                w_v[r] = e / jnp.sum(e)
                idx_v[pl.ds(pl.multiple_of(r * 16, 16), 16)] = (
                    jnp.minimum(ti, K - 1))
            return 0

        with jax.named_scope("sc_merge"):
            lax.fori_loop(0, RPW // ROWG, merge_rows, 0)

        with jax.named_scope("sc_table_wait"):
            table_cp.wait()

        def crow(r, _):
            tiv = idx_v[pl.ds(pl.multiple_of(r * 16, 16), 16)]
            wvec = w_v[r]
            accs = [jnp.zeros((16,), jnp.float32) for _ in range(OUT // 16)]
            for j in range(16 - TOP_K, 16):
                kj = _lane_bcast(tiv, j)
                wj = _lane_bcast(wvec, j)
                krow = lax.shift_right_logical(kj, 1)
                kcol = jnp.bitwise_and(kj, 1) * OUT
                for c in range(OUT // 16):
                    col = kcol + c * 16 + lane
                    v = plsc.load_gather(table_v, [krow, col])
                    accs[c] = accs[c] + wj * v
            for c in range(OUT // 16):
                acc_v[r, pl.ds(c * 16, 16)] = accs[c]
            return 0

        with jax.named_scope("sc_combine"):
            lax.fori_loop(0, RPW, crow, 0)
        with jax.named_scope("sc_dma_out"):
            pltpu.sync_copy(acc_v, out_hbm.at[pl.ds(base, RPW)])

    return sc_top10


_sc_combine = _make_sc_combine()


@jax.jit
def kernel(x, mean, stddev, outputs):
    lp, gid = _distances(x, mean, stddev)
    outputs2 = outputs.reshape(K // 2, 2 * OUT)
    return _sc_combine(lp, gid, outputs2)


# speedup vs baseline: 1.7340x; 1.0743x over previous
import functools

import jax
import jax.numpy as jnp
from jax import lax
from jax.experimental import pallas as pl
from jax.experimental.pallas import tpu as pltpu
from jax.experimental.pallas import tpu_sc as plsc

B = 1024
K = 1000
D = 128
OUT = 64
TOP_K = 10
KP = 1024
BB = 256
NW = 32
RPW = B // NW
NG = 128
GSZ = KP // NG
ROWG = 4

_NEG = -3.0e38


def _lp_body(x_ref, mean_ref, stddev_ref, lp_ref, gid_ref,
             w1_s, w2_s, bias_s):
    @pl.when(pl.program_id(0) == 0)
    def _prep():
        mean = mean_ref[...]
        std = stddev_ref[...]
        iv = 1.0 / (std * std)
        w2 = mean * iv
        w1_s[...] = -0.5 * iv
        w2_s[...] = w2
        bias_s[...] = (jnp.sum(-0.5 * mean * w2 - jnp.log(std), axis=1)
                       - 0.5 * D * jnp.log(2.0 * jnp.pi))[None, :]

    x = x_ref[...]
    t1 = jax.lax.dot_general(x * x, w1_s[...], (((1,), (1,)), ((), ())),
                             preferred_element_type=jnp.float32,
                             precision=jax.lax.Precision.HIGHEST)
    t2 = jax.lax.dot_general(x, w2_s[...], (((1,), (1,)), ((), ())),
                             preferred_element_type=jnp.float32,
                             precision=jax.lax.Precision.HIGHEST)
    lp = t1 + t2 + bias_s[...]
    lp = jnp.concatenate(
        [lp, jnp.full((BB, KP - K), _NEG, jnp.float32)], axis=1)
    lp_ref[...] = lp

    m = jnp.max(lp.reshape(BB, GSZ, NG), axis=1)
    iota = jax.lax.broadcasted_iota(jnp.int32, (BB, NG), 1)
    gid = jnp.zeros((BB, NG), jnp.int32)
    for i in range(TOP_K):
        cur = jnp.max(m, axis=1, keepdims=True)
        ismax = m == cur
        first = jnp.min(jnp.where(ismax, iota, NG), axis=1, keepdims=True)
        gid = jnp.where(iota == i, first, gid)
        m = jnp.where(iota == first, _NEG, m)
    gid_ref[...] = gid


def _distances(x, mean, stddev):
    return pl.pallas_call(
        _lp_body,
        grid=(B // BB,),
        in_specs=[
            pl.BlockSpec((BB, D), lambda i: (i, 0)),
            pl.BlockSpec((K, D), lambda i: (0, 0)),
            pl.BlockSpec((K, D), lambda i: (0, 0)),
        ],
        out_specs=[
            pl.BlockSpec((BB, KP), lambda i: (i, 0)),
            pl.BlockSpec((BB, NG), lambda i: (i, 0)),
        ],
        out_shape=[
            jax.ShapeDtypeStruct((B, KP), jnp.float32),
            jax.ShapeDtypeStruct((B, NG), jnp.int32),
        ],
        scratch_shapes=[
            pltpu.VMEM((K, D), jnp.float32),
            pltpu.VMEM((K, D), jnp.float32),
            pltpu.VMEM((1, K), jnp.float32),
        ],
        compiler_params=pltpu.CompilerParams(
            dimension_semantics=("arbitrary",)),
    )(x, mean, stddev)


def _lane_bcast(v, j):
    idx = jnp.full((16, 1), j, jnp.int32)
    dn = lax.GatherDimensionNumbers(
        offset_dims=(), collapsed_slice_dims=(0,), start_index_map=(0,))
    return lax.gather(v, idx, dn, (1,),
                      mode=lax.GatherScatterMode.PROMISE_IN_BOUNDS)


def _make_sc_combine():
    mesh = plsc.VectorSubcoreMesh(core_axis_name="c", subcore_axis_name="s",
                                  num_cores=2, num_subcores=16)

    @functools.partial(
        pl.kernel,
        out_type=jax.ShapeDtypeStruct((B, OUT), jnp.float32),
        mesh=mesh,
        scratch_types=[
            pltpu.VMEM((RPW, KP), jnp.float32),
            pltpu.VMEM((RPW, NG), jnp.int32),
            pltpu.VMEM((RPW * 16,), jnp.int32),
            pltpu.VMEM((RPW, 16), jnp.float32),
            pltpu.VMEM((K // 2, 2 * OUT), jnp.float32),
            pltpu.VMEM_SHARED((K // 2, 2 * OUT), jnp.float32),
            pltpu.VMEM((RPW, OUT), jnp.float32),
            pltpu.SemaphoreType.DMA,
        ],
        compiler_params=pltpu.CompilerParams(needs_layout_passes=False),
    )
    def sc_top10(lp_hbm, gid_hbm, outputs_hbm, out_hbm,
                 rows_v, gid_v, idx_v, w_v, table_v, table_sh, acc_v, sem):
        wid = lax.axis_index("s") * 2 + lax.axis_index("c")
        base = pl.multiple_of(wid * RPW, RPW)
        with jax.named_scope("sc_stage_table"):
            @pl.when(lax.axis_index("s") == 0)
            def _stage():
                pltpu.sync_copy(outputs_hbm, table_sh)
            plsc.subcore_barrier()
        with jax.named_scope("sc_dma_in"):
            pltpu.sync_copy(lp_hbm.at[pl.ds(base, RPW)], rows_v)
            pltpu.sync_copy(gid_hbm.at[pl.ds(base, RPW)], gid_v)
        table_cp = pltpu.async_copy(table_sh, table_v, sem)

        lane = lax.broadcasted_iota(jnp.int32, (16,), 0)
        lane8 = jnp.bitwise_and(lane, GSZ - 1)

        def merge_rows(g, _):
            gbase = pl.multiple_of(g * ROWG, ROWG)
            for q in range(ROWG):
                r = gbase + q
                rfull = jnp.full((16,), r, jnp.int32)
                gids = gid_v[r, pl.ds(0, 16)]
                hi = lane >= GSZ

                def pair(j2):
                    ga = _lane_bcast(gids, 2 * j2)
                    gb = _lane_bcast(gids, 2 * j2 + 1)
                    gi = jnp.where(hi, gb, ga) + NG * lane8
                    return plsc.load_gather(rows_v, [rfull, gi]), gi

                v, gi = pair(0)
                tv, ti = plsc.sort_key_val(v, gi)
                for j2 in range(1, TOP_K // 2):
                    v, gi = pair(j2)
                    sv, si = plsc.sort_key_val(v, gi, descending=True)
                    m = tv >= sv
                    nv = jnp.where(m, tv, sv)
                    ni = jnp.where(m, ti, si)
                    tv, ti = plsc.sort_key_val(nv, ni)
                m0 = jnp.max(tv)
                e = jnp.where(lane >= 16 - TOP_K, jnp.exp(tv - m0), 0.0)
                w_v[r] = e / jnp.sum(e)
                idx_v[pl.ds(pl.multiple_of(r * 16, 16), 16)] = (
                    jnp.minimum(ti, K - 1))
            return 0

        with jax.named_scope("sc_merge"):
            lax.fori_loop(0, RPW // ROWG, merge_rows, 0)

        with jax.named_scope("sc_table_wait"):
            table_cp.wait()

        def crow(r, _):
            tiv = idx_v[pl.ds(pl.multiple_of(r * 16, 16), 16)]
            wvec = w_v[r]
            accs = [jnp.zeros((16,), jnp.float32) for _ in range(OUT // 16)]
            for j in range(16 - TOP_K, 16):
                kj = _lane_bcast(tiv, j)
                wj = _lane_bcast(wvec, j)
                krow = lax.shift_right_logical(kj, 1)
                kcol = jnp.bitwise_and(kj, 1) * OUT
                for c in range(OUT // 16):
                    col = kcol + c * 16 + lane
                    v = plsc.load_gather(table_v, [krow, col])
                    accs[c] = accs[c] + wj * v
            for c in range(OUT // 16):
                acc_v[r, pl.ds(c * 16, 16)] = accs[c]
            return 0

        with jax.named_scope("sc_combine"):
            lax.fori_loop(0, RPW, crow, 0)
        with jax.named_scope("sc_dma_out"):
            pltpu.sync_copy(acc_v, out_hbm.at[pl.ds(base, RPW)])

    return sc_top10


_sc_combine = _make_sc_combine()


@jax.jit
def kernel(x, mean, stddev, outputs):
    lp, gid = _distances(x, mean, stddev)
    outputs2 = outputs.reshape(K // 2, 2 * OUT)
    return _sc_combine(lp, gid, outputs2)


# speedup vs baseline: 1.9580x; 1.1292x over previous
import functools

import jax
import jax.numpy as jnp
from jax import lax
from jax.experimental import pallas as pl
from jax.experimental.pallas import tpu as pltpu
from jax.experimental.pallas import tpu_sc as plsc

B = 1024
K = 1000
D = 128
OUT = 64
TOP_K = 10
KP = 1024
BB = 1024
NW = 32
RPW = B // NW
NG = 128
GSZ = KP // NG
ROWG = 4

_NEG = -3.0e38


def _lp_body(x_ref, mean_ref, stddev_ref, lp_ref, gid_ref,
             w1_s, w2_s, bias_s):
    @pl.when(pl.program_id(0) == 0)
    def _prep():
        mean = mean_ref[...]
        std = stddev_ref[...]
        iv = 1.0 / (std * std)
        w2 = mean * iv
        w1_s[...] = -0.5 * iv
        w2_s[...] = w2
        bias_s[...] = (jnp.sum(-0.5 * mean * w2 - jnp.log(std), axis=1)
                       - 0.5 * D * jnp.log(2.0 * jnp.pi))[None, :]

    x = x_ref[...]
    t1 = jax.lax.dot_general(x * x, w1_s[...], (((1,), (1,)), ((), ())),
                             preferred_element_type=jnp.float32,
                             precision=jax.lax.Precision.HIGHEST)
    t2 = jax.lax.dot_general(x, w2_s[...], (((1,), (1,)), ((), ())),
                             preferred_element_type=jnp.float32,
                             precision=jax.lax.Precision.HIGHEST)
    lp = t1 + t2 + bias_s[...]
    lp = jnp.concatenate(
        [lp, jnp.full((BB, KP - K), _NEG, jnp.float32)], axis=1)
    lp_ref[...] = lp

    m = jnp.max(lp.reshape(BB, GSZ, NG), axis=1)
    iota = jax.lax.broadcasted_iota(jnp.int32, (BB, NG), 1)
    gid = jnp.zeros((BB, NG), jnp.int32)
    for i in range(TOP_K):
        cur = jnp.max(m, axis=1, keepdims=True)
        ismax = m == cur
        first = jnp.min(jnp.where(ismax, iota, NG), axis=1, keepdims=True)
        gid = jnp.where(iota == i, first, gid)
        m = jnp.where(iota == first, _NEG, m)
    gid_ref[...] = gid


def _distances(x, mean, stddev):
    return pl.pallas_call(
        _lp_body,
        grid=(B // BB,),
        in_specs=[
            pl.BlockSpec((BB, D), lambda i: (i, 0)),
            pl.BlockSpec((K, D), lambda i: (0, 0)),
            pl.BlockSpec((K, D), lambda i: (0, 0)),
        ],
        out_specs=[
            pl.BlockSpec((BB, KP), lambda i: (i, 0)),
            pl.BlockSpec((BB, NG), lambda i: (i, 0)),
        ],
        out_shape=[
            jax.ShapeDtypeStruct((B, KP), jnp.float32),
            jax.ShapeDtypeStruct((B, NG), jnp.int32),
        ],
        scratch_shapes=[
            pltpu.VMEM((K, D), jnp.float32),
            pltpu.VMEM((K, D), jnp.float32),
            pltpu.VMEM((1, K), jnp.float32),
        ],
        compiler_params=pltpu.CompilerParams(
            dimension_semantics=("arbitrary",)),
    )(x, mean, stddev)


def _lane_bcast(v, j):
    idx = jnp.full((16, 1), j, jnp.int32)
    dn = lax.GatherDimensionNumbers(
        offset_dims=(), collapsed_slice_dims=(0,), start_index_map=(0,))
    return lax.gather(v, idx, dn, (1,),
                      mode=lax.GatherScatterMode.PROMISE_IN_BOUNDS)


def _make_sc_combine():
    mesh = plsc.VectorSubcoreMesh(core_axis_name="c", subcore_axis_name="s",
                                  num_cores=2, num_subcores=16)

    @functools.partial(
        pl.kernel,
        out_type=jax.ShapeDtypeStruct((B, OUT), jnp.float32),
        mesh=mesh,
        scratch_types=[
            pltpu.VMEM((RPW, KP), jnp.float32),
            pltpu.VMEM((RPW, NG), jnp.int32),
            pltpu.VMEM((RPW * 16,), jnp.int32),
            pltpu.VMEM((RPW, 16), jnp.float32),
            pltpu.VMEM((K // 2, 2 * OUT), jnp.float32),
            pltpu.VMEM_SHARED((K // 2, 2 * OUT), jnp.float32),
            pltpu.VMEM((RPW, OUT), jnp.float32),
            pltpu.SemaphoreType.DMA,
        ],
        compiler_params=pltpu.CompilerParams(needs_layout_passes=False),
    )
    def sc_top10(lp_hbm, gid_hbm, outputs_hbm, out_hbm,
                 rows_v, gid_v, idx_v, w_v, table_v, table_sh, acc_v, sem):
        wid = lax.axis_index("s") * 2 + lax.axis_index("c")
        base = pl.multiple_of(wid * RPW, RPW)
        with jax.named_scope("sc_stage_table"):
            @pl.when(lax.axis_index("s") == 0)
            def _stage():
                pltpu.sync_copy(outputs_hbm, table_sh)
            plsc.subcore_barrier()
        with jax.named_scope("sc_dma_in"):
            pltpu.sync_copy(lp_hbm.at[pl.ds(base, RPW)], rows_v)
            pltpu.sync_copy(gid_hbm.at[pl.ds(base, RPW)], gid_v)
        table_cp = pltpu.async_copy(table_sh, table_v, sem)

        lane = lax.broadcasted_iota(jnp.int32, (16,), 0)
        lane8 = jnp.bitwise_and(lane, GSZ - 1)

        def merge_rows(g, _):
            gbase = pl.multiple_of(g * ROWG, ROWG)
            for q in range(ROWG):
                r = gbase + q
                rfull = jnp.full((16,), r, jnp.int32)
                gids = gid_v[r, pl.ds(0, 16)]
                hi = lane >= GSZ

                def pair(j2):
                    ga = _lane_bcast(gids, 2 * j2)
                    gb = _lane_bcast(gids, 2 * j2 + 1)
                    gi = jnp.where(hi, gb, ga) + NG * lane8
                    return plsc.load_gather(rows_v, [rfull, gi]), gi

                v, gi = pair(0)
                tv, ti = plsc.sort_key_val(v, gi)
                for j2 in range(1, TOP_K // 2):
                    v, gi = pair(j2)
                    sv, si = plsc.sort_key_val(v, gi, descending=True)
                    m = tv >= sv
                    nv = jnp.where(m, tv, sv)
                    ni = jnp.where(m, ti, si)
                    tv, ti = plsc.sort_key_val(nv, ni)
                m0 = jnp.max(tv)
                e = jnp.where(lane >= 16 - TOP_K, jnp.exp(tv - m0), 0.0)
                w_v[r] = e / jnp.sum(e)
                idx_v[pl.ds(pl.multiple_of(r * 16, 16), 16)] = (
                    jnp.minimum(ti, K - 1))
            return 0

        with jax.named_scope("sc_merge"):
            lax.fori_loop(0, RPW // ROWG, merge_rows, 0)

        with jax.named_scope("sc_table_wait"):
            table_cp.wait()

        def crow(r, _):
            tiv = idx_v[pl.ds(pl.multiple_of(r * 16, 16), 16)]
            wvec = w_v[r]
            accs = [jnp.zeros((16,), jnp.float32) for _ in range(OUT // 16)]
            for j in range(16 - TOP_K, 16):
                kj = _lane_bcast(tiv, j)
                wj = _lane_bcast(wvec, j)
                krow = lax.shift_right_logical(kj, 1)
                kcol = jnp.bitwise_and(kj, 1) * OUT
                for c in range(OUT // 16):
                    col = kcol + c * 16 + lane
                    v = plsc.load_gather(table_v, [krow, col])
                    accs[c] = accs[c] + wj * v
            for c in range(OUT // 16):
                acc_v[r, pl.ds(c * 16, 16)] = accs[c]
            return 0

        with jax.named_scope("sc_combine"):
            lax.fori_loop(0, RPW, crow, 0)
        with jax.named_scope("sc_dma_out"):
            pltpu.sync_copy(acc_v, out_hbm.at[pl.ds(base, RPW)])

    return sc_top10


_sc_combine = _make_sc_combine()


@jax.jit
def kernel(x, mean, stddev, outputs):
    lp, gid = _distances(x, mean, stddev)
    outputs2 = outputs.reshape(K // 2, 2 * OUT)
    return _sc_combine(lp, gid, outputs2)
